# Initial kernel scaffold; baseline (speedup 1.0000x reference)
import jax
import jax.numpy as jnp
from jax.experimental import pallas as pl


def kernel(x, W, b, rW1, rb1, rW2, rb2, gates, A, B):
    raise NotImplementedError("write your pallas kernel here")


# speedup vs baseline: 4.5309x; 4.5309x over previous
import jax
import jax.numpy as jnp
from jax.experimental import pallas as pl
from jax.experimental.pallas import tpu as pltpu

_BM = 512


def _fused_body(x_ref, Wb_ref, b_ref, rW1_ref, rb1_ref, rW2_ref, rb2g_ref,
                Ab_ref, Bb_ref, E_ref, o_ref):
    x = x_ref[...]
    xb = x.astype(jnp.bfloat16)

    h = jax.lax.dot_general(x, rW1_ref[...], (((1,), (1,)), ((), ())),
                            preferred_element_type=jnp.float32)
    h = h + rb1_ref[...]
    h = h * jax.nn.sigmoid(h)
    logits = jax.lax.dot_general(h, rW2_ref[...], (((1,), (1,)), ((), ())),
                                 preferred_element_type=jnp.float32)
    logits = logits + rb2g_ref[...]

    idx = jax.lax.broadcasted_iota(jnp.int32, logits.shape, 1)
    m1 = jnp.max(logits, axis=-1, keepdims=True)
    i1 = jnp.min(jnp.where(logits == m1, idx, logits.shape[-1]),
                 axis=-1, keepdims=True)
    masked = jnp.where(idx == i1, -jnp.inf, logits)
    m2 = jnp.max(masked, axis=-1, keepdims=True)
    i2 = jnp.min(jnp.where(masked == m2, idx, logits.shape[-1]),
                 axis=-1, keepdims=True)
    p1 = jax.nn.sigmoid(m1 - m2)
    coeff = jnp.where(idx == i1, p1, jnp.where(idx == i2, 1.0 - p1, 0.0))

    C = jnp.dot(coeff, E_ref[...], preferred_element_type=jnp.float32)

    u = jax.lax.dot_general(xb, Ab_ref[...], (((1,), (1,)), ((), ())),
                            preferred_element_type=jnp.float32)
    uw = (u * C).astype(jnp.bfloat16)
    delta = jnp.dot(uw, Bb_ref[...], preferred_element_type=jnp.float32)

    base = jax.lax.dot_general(xb, Wb_ref[...], (((1,), (1,)), ((), ())),
                               preferred_element_type=jnp.float32)

    o_ref[...] = base + delta + b_ref[...]


def kernel(x, W, b, rW1, rb1, rW2, rb2, gates, A, B):
    n_tokens, d_in = x.shape
    d_out = W.shape[0]
    num_experts, rank = A.shape[0], A.shape[1]
    r_hid = rW1.shape[0]

    Wb = W.astype(jnp.bfloat16)
    Ab = A.reshape(num_experts * rank, d_in).astype(jnp.bfloat16)
    Bb = jnp.transpose(B, (0, 2, 1)).reshape(
        num_experts * rank, d_out).astype(jnp.bfloat16)
    rW2e = rW2[:num_experts]
    rb2g = (rb2[:num_experts] + gates).reshape(1, num_experts)
    E = jnp.kron(jnp.eye(num_experts, dtype=jnp.float32),
                 jnp.ones((1, rank), dtype=jnp.float32))

    bm = _BM
    grid = (n_tokens // bm,)

    full = lambda shape: pl.BlockSpec(shape, lambda i: (0,) * len(shape))
    out = pl.pallas_call(
        _fused_body,
        grid=grid,
        in_specs=[
            pl.BlockSpec((bm, d_in), lambda i: (i, 0)),
            full((d_out, d_in)),
            full((1, d_out)),
            full((r_hid, d_in)),
            full((1, r_hid)),
            full((num_experts, r_hid)),
            full((1, num_experts)),
            full((num_experts * rank, d_in)),
            full((num_experts * rank, d_out)),
            full((num_experts, num_experts * rank)),
        ],
        out_specs=pl.BlockSpec((bm, d_out), lambda i: (i, 0)),
        out_shape=jax.ShapeDtypeStruct((n_tokens, d_out), jnp.float32),
        compiler_params=pltpu.CompilerParams(
            dimension_semantics=("arbitrary",)),
    )(x, Wb, b.reshape(1, d_out), rW1, rb1.reshape(1, r_hid),
      rW2e, rb2g, Ab, Bb, E)
    return out


# speedup vs baseline: 4.6914x; 1.0354x over previous
import jax
import jax.numpy as jnp
from jax.experimental import pallas as pl
from jax.experimental.pallas import tpu as pltpu

_BM = 512


def _fused_body(x_ref, Wb_ref, b_ref, rb1_ref, rW2_ref, rb2g_ref,
                ARb_ref, Bb_ref, E_ref, o_ref):
    nr = ARb_ref.shape[0] - rb1_ref.shape[1]
    x = x_ref[...]
    xb = x.astype(jnp.bfloat16)

    v = jax.lax.dot_general(xb, ARb_ref[...], (((1,), (1,)), ((), ())),
                            preferred_element_type=jnp.float32)
    u = v[:, :nr]
    h = v[:, nr:] + rb1_ref[...]
    h = h * jax.nn.sigmoid(h)
    logits = jax.lax.dot_general(h.astype(jnp.bfloat16), rW2_ref[...],
                                 (((1,), (1,)), ((), ())),
                                 preferred_element_type=jnp.float32)
    logits = logits + rb2g_ref[...]

    idx = jax.lax.broadcasted_iota(jnp.int32, logits.shape, 1)
    m1 = jnp.max(logits, axis=-1, keepdims=True)
    i1 = jnp.min(jnp.where(logits == m1, idx, logits.shape[-1]),
                 axis=-1, keepdims=True)
    masked = jnp.where(idx == i1, -jnp.inf, logits)
    m2 = jnp.max(masked, axis=-1, keepdims=True)
    i2 = jnp.min(jnp.where(masked == m2, idx, logits.shape[-1]),
                 axis=-1, keepdims=True)
    p1 = jax.nn.sigmoid(m1 - m2)
    coeff = jnp.where(idx == i1, p1, jnp.where(idx == i2, 1.0 - p1, 0.0))

    C = jnp.dot(coeff, E_ref[...], preferred_element_type=jnp.float32)
    uw = (u * C).astype(jnp.bfloat16)
    delta = jnp.dot(uw, Bb_ref[...], preferred_element_type=jnp.float32)

    base = jax.lax.dot_general(xb, Wb_ref[...], (((1,), (1,)), ((), ())),
                               preferred_element_type=jnp.float32)

    o_ref[...] = base + delta + b_ref[...]


def kernel(x, W, b, rW1, rb1, rW2, rb2, gates, A, B):
    n_tokens, d_in = x.shape
    d_out = W.shape[0]
    num_experts, rank = A.shape[0], A.shape[1]
    r_hid = rW1.shape[0]

    Wb = W.astype(jnp.bfloat16)
    ARb = jnp.concatenate(
        [A.reshape(num_experts * rank, d_in), rW1],
        axis=0).astype(jnp.bfloat16)
    Bb = jnp.transpose(B, (0, 2, 1)).reshape(
        num_experts * rank, d_out).astype(jnp.bfloat16)
    rW2e = rW2[:num_experts].astype(jnp.bfloat16)
    rb2g = (rb2[:num_experts] + gates).reshape(1, num_experts)
    E = jnp.kron(jnp.eye(num_experts, dtype=jnp.float32),
                 jnp.ones((1, rank), dtype=jnp.float32))

    bm = _BM
    grid = (n_tokens // bm,)

    full = lambda shape: pl.BlockSpec(shape, lambda i: (0,) * len(shape))
    out = pl.pallas_call(
        _fused_body,
        grid=grid,
        in_specs=[
            pl.BlockSpec((bm, d_in), lambda i: (i, 0)),
            full((d_out, d_in)),
            full((1, d_out)),
            full((1, r_hid)),
            full((num_experts, r_hid)),
            full((1, num_experts)),
            full((num_experts * rank + r_hid, d_in)),
            full((num_experts * rank, d_out)),
            full((num_experts, num_experts * rank)),
        ],
        out_specs=pl.BlockSpec((bm, d_out), lambda i: (i, 0)),
        out_shape=jax.ShapeDtypeStruct((n_tokens, d_out), jnp.float32),
        compiler_params=pltpu.CompilerParams(
            dimension_semantics=("arbitrary",)),
    )(x, Wb, b.reshape(1, d_out), rb1.reshape(1, r_hid),
      rW2e, rb2g, ARb, Bb, E)
    return out
